# baseline (speedup 1.0000x reference)
import functools

import jax
import jax.numpy as jnp
from jax import lax
from jax.experimental import pallas as pl
from jax.experimental.pallas import tpu as pltpu
from jax.experimental.pallas import tpu_sc as plsc

N = 10000
E = 320000
D = 128
EDGE_DIM = 16

NC = 2
NS = 16
NW = NC * NS
SUB = 128
ISTEP = 1024
REG = 256
KC = 10
CE = NW * ISTEP
EPAD = KC * CE
SCH = KC // 2
SCHUNK = 256
NPAD = 240
NACC = N + NPAD
ZROWS = NACC // NS


@functools.lru_cache(maxsize=None)
def _sc_mesh():
    return plsc.VectorSubcoreMesh(core_axis_name="c", subcore_axis_name="s",
                                  num_cores=NC, num_subcores=NS)


def _gather_body(chunk, nf_hbm, src_hbm, dst_hbm, src_out, dst_out,
                 idx_v, rows_v, gsem, osem0, osem1):
    c = lax.axis_index("c")
    s = lax.axis_index("s")
    wid = s * NC + c
    base_g = chunk * (CE // SUB) + wid * (ISTEP // SUB)

    pltpu.sync_copy(src_hbm.at[pl.ds(base_g, ISTEP // SUB)],
                    idx_v.at[pl.ds(0, ISTEP // SUB)])
    pltpu.sync_copy(dst_hbm.at[pl.ds(base_g, ISTEP // SUB)],
                    idx_v.at[pl.ds(ISTEP // SUB, ISTEP // SUB)])

    osems = (osem0, osem1)
    ocopies = []
    k = 0
    for t, out_hbm in ((0, src_out), (1, dst_out)):
        for r in range(ISTEP // REG):
            buf = k % 2
            if k >= 2:
                ocopies[k - 2].wait()
            descs = []
            for j in range(REG // SUB):
                g = t * (ISTEP // SUB) + r * (REG // SUB) + j
                descs.append(pltpu.async_copy(
                    nf_hbm.at[idx_v.at[g]],
                    rows_v.at[pl.ds(buf * REG + j * SUB, SUB)],
                    gsem,
                ))
            for dsc in descs:
                dsc.wait()
            ocopies.append(pltpu.async_copy(
                rows_v.at[pl.ds(buf * REG, REG)],
                out_hbm.at[pl.ds(wid * ISTEP + r * REG, REG)],
                osems[buf],
            ))
            k += 1
    ocopies[-2].wait()
    ocopies[-1].wait()


@functools.lru_cache(maxsize=None)
def _gather_call(chunk):
    return pl.kernel(
        functools.partial(_gather_body, chunk),
        out_type=(
            jax.ShapeDtypeStruct((CE, D), jnp.float32),
            jax.ShapeDtypeStruct((CE, D), jnp.float32),
        ),
        mesh=_sc_mesh(),
        scratch_types=[
            pltpu.VMEM((2 * ISTEP // SUB, SUB), jnp.int32),
            pltpu.VMEM((2 * REG, D), jnp.float32),
            pltpu.SemaphoreType.DMA,
            pltpu.SemaphoreType.DMA,
            pltpu.SemaphoreType.DMA,
        ],
        name=f"mpn_gather_{chunk}",
    )


def _scatter_body(half, dst_hbm, zeros_hbm, *rest):
    msgs = rest[:SCH]
    part_out = rest[SCH]
    acc_sh, idx_v, rows_v = rest[SCH + 1:]
    c = lax.axis_index("c")
    s = lax.axis_index("s")
    pltpu.sync_copy(zeros_hbm.at[pl.ds(s * ZROWS, ZROWS)],
                    acc_sh.at[pl.ds(s * ZROWS, ZROWS)])
    plsc.subcore_barrier()

    wid = c * NS + s
    for k in range(SCH):
        base_g = (half * SCH + k) * (CE // SUB) + wid * (ISTEP // SUB)
        pltpu.sync_copy(dst_hbm.at[pl.ds(base_g, ISTEP // SUB)], idx_v)
        for h in range(ISTEP // SCHUNK):
            pltpu.sync_copy(
                msgs[k].at[pl.ds(wid * ISTEP + h * SCHUNK, SCHUNK)], rows_v)
            for j in range(SCHUNK // SUB):
                pltpu.sync_copy(
                    rows_v.at[pl.ds(j * SUB, SUB)],
                    acc_sh.at[idx_v.at[h * (SCHUNK // SUB) + j]],
                    add=True,
                )
    plsc.subcore_barrier()
    pltpu.sync_copy(acc_sh.at[pl.ds(s * ZROWS, ZROWS)],
                    part_out.at[c].at[pl.ds(s * ZROWS, ZROWS)])


@functools.lru_cache(maxsize=None)
def _scatter_call(half):
    return pl.kernel(
        functools.partial(_scatter_body, half),
        out_type=jax.ShapeDtypeStruct((NC, NACC, D), jnp.float32),
        mesh=_sc_mesh(),
        scratch_types=[
            pltpu.VMEM_SHARED((NACC, D), jnp.float32),
            pltpu.VMEM((ISTEP // SUB, SUB), jnp.int32),
            pltpu.VMEM((SCHUNK, D), jnp.float32),
        ],
        name=f"mpn_scatter_{half}",
    )


BE = 1024


def _mlp_body(src_ref, dst_ref, ef_ref, w1s_ref, w1d_ref, w1f_ref, b1e_ref,
              w2e_ref, b2e_ref, w1nd_ref, w1nm_ref, b1n_ref, w2n_ref, b2n_ref,
              out_ref):
    bf = jnp.bfloat16
    mm = functools.partial(jax.lax.dot_general,
                           dimension_numbers=(((1,), (0,)), ((), ())),
                           preferred_element_type=jnp.float32)
    src = src_ref[...].astype(bf)
    dst = dst_ref[...].astype(bf)
    ef = ef_ref[...].astype(bf)
    h = (mm(src, w1s_ref[...].astype(bf)) + mm(dst, w1d_ref[...].astype(bf))
         + mm(ef, w1f_ref[...].astype(bf)) + b1e_ref[...])
    h = jnp.maximum(h, 0.0).astype(bf)
    msg = jnp.maximum(mm(h, w2e_ref[...].astype(bf)) + b2e_ref[...], 0.0)
    msgb = msg.astype(bf)
    g = (mm(dst, w1nd_ref[...].astype(bf)) + mm(msgb, w1nm_ref[...].astype(bf))
         + b1n_ref[...])
    g = jnp.maximum(g, 0.0).astype(bf)
    out_ref[...] = jnp.maximum(mm(g, w2n_ref[...].astype(bf)) + b2n_ref[...], 0.0)


def _mlp_call(src_feat, dst_feat, ef, weights):
    eb = lambda i: (i, 0)
    wb = lambda i: (0, 0)
    wb3 = lambda i: (0, 0, 0)
    return pl.pallas_call(
        _mlp_body,
        grid=(CE // BE,),
        in_specs=[
            pl.BlockSpec((BE, D), eb),
            pl.BlockSpec((BE, D), eb),
            pl.BlockSpec((BE, EDGE_DIM), eb),
            pl.BlockSpec((D, 32), wb),
            pl.BlockSpec((D, 32), wb),
            pl.BlockSpec((EDGE_DIM, 32), wb),
            pl.BlockSpec((1, 32), wb),
            pl.BlockSpec((32, D), wb),
            pl.BlockSpec((1, D), wb),
            pl.BlockSpec((D, 64), wb),
            pl.BlockSpec((D, 64), wb),
            pl.BlockSpec((1, 64), wb),
            pl.BlockSpec((64, D), wb),
            pl.BlockSpec((1, D), wb),
        ],
        out_specs=pl.BlockSpec((BE, D), eb),
        out_shape=jax.ShapeDtypeStruct((CE, D), jnp.float32),
        name="mpn_mlp",
    )(src_feat, dst_feat, ef, *weights)


BN = 1000


def _combine_body(nf_ref, pa_ref, pb_ref, out_ref):
    out_ref[...] = (nf_ref[...] + (pa_ref[0, 0] + pa_ref[1, 0])
                    + (pb_ref[0, 0] + pb_ref[1, 0]))


def _combine_call(nf, parts0, parts1):
    return pl.pallas_call(
        _combine_body,
        grid=(N // BN,),
        in_specs=[
            pl.BlockSpec((BN, D), lambda i: (i, 0)),
            pl.BlockSpec((2, 1, BN, D), lambda i: (0, 0, i, 0)),
            pl.BlockSpec((2, 1, BN, D), lambda i: (0, 0, i, 0)),
        ],
        out_specs=pl.BlockSpec((BN, D), lambda i: (i, 0)),
        out_shape=jax.ShapeDtypeStruct((N, D), jnp.float32),
        name="mpn_combine",
    )(nf, parts0.reshape(NC, 1, NACC, D), parts1.reshape(NC, 1, NACC, D))


def kernel(node_features, edge_features, edge_index, W1e, b1e, W2e, b2e,
           W1n, b1n, W2n, b2n):
    src = edge_index[0].astype(jnp.int32)
    dst = edge_index[1].astype(jnp.int32)
    pad = EPAD - E
    ar = jnp.arange(pad, dtype=jnp.int32)
    pad_gather = ar % N
    pad_scatter = N + (ar % NPAD)

    src2d = jnp.concatenate([src, pad_gather]).reshape(EPAD // SUB, SUB)
    dstg2d = jnp.concatenate([dst, pad_gather]).reshape(EPAD // SUB, SUB)
    dsts2d = jnp.concatenate([dst, pad_scatter]).reshape(EPAD // SUB, SUB)
    ef_pad = jnp.concatenate(
        [edge_features, jnp.zeros((pad, EDGE_DIM), jnp.float32)])
    zeros = jnp.zeros((NACC, D), jnp.float32)

    weights = (
        W1e[:D], W1e[D:2 * D], W1e[2 * D:], b1e.reshape(1, 32),
        W2e, b2e.reshape(1, D),
        W1n[:D], W1n[D:], b1n.reshape(1, 64),
        W2n, b2n.reshape(1, D),
    )

    msgs = []
    for c in range(KC):
        src_feat, dst_feat = _gather_call(c)(node_features, src2d, dstg2d)
        ef_c = jax.lax.slice_in_dim(ef_pad, c * CE, (c + 1) * CE)
        msgs.append(_mlp_call(src_feat, dst_feat, ef_c, weights))

    parts0 = _scatter_call(0)(dsts2d, zeros, *msgs[:SCH])
    parts1 = _scatter_call(1)(dsts2d, zeros, *msgs[SCH:])
    return _combine_call(node_features, parts0, parts1)


# speedup vs baseline: 1.1042x; 1.1042x over previous
import functools

import jax
import jax.numpy as jnp
from jax import lax
from jax.experimental import pallas as pl
from jax.experimental.pallas import tpu as pltpu
from jax.experimental.pallas import tpu_sc as plsc

N = 10000
E = 320000
D = 128
EDGE_DIM = 16

NC = 2
NS = 16
NW = NC * NS
SUB = 128
ISTEP = 1024
REG = 256
KC = 10
CE = NW * ISTEP
EPAD = KC * CE
SCH = KC // 2
SCHUNK = 256
NPAD = 240
NACC = N + NPAD
ZROWS = NACC // NS


@functools.lru_cache(maxsize=None)
def _sc_mesh():
    return plsc.VectorSubcoreMesh(core_axis_name="c", subcore_axis_name="s",
                                  num_cores=NC, num_subcores=NS)


def _gather_body(chunk, nf_hbm, src_hbm, dst_hbm, src_out, dst_out,
                 idx_v, rows_v, gsem, osem0, osem1):
    c = lax.axis_index("c")
    s = lax.axis_index("s")
    wid = s * NC + c
    base_g = chunk * (CE // SUB) + wid * (ISTEP // SUB)

    pltpu.sync_copy(src_hbm.at[pl.ds(base_g, ISTEP // SUB)],
                    idx_v.at[pl.ds(0, ISTEP // SUB)])
    pltpu.sync_copy(dst_hbm.at[pl.ds(base_g, ISTEP // SUB)],
                    idx_v.at[pl.ds(ISTEP // SUB, ISTEP // SUB)])

    osems = (osem0, osem1)
    ocopies = []
    k = 0
    for t, out_hbm in ((0, src_out), (1, dst_out)):
        for r in range(ISTEP // REG):
            buf = k % 2
            if k >= 2:
                ocopies[k - 2].wait()
            descs = []
            for j in range(REG // SUB):
                g = t * (ISTEP // SUB) + r * (REG // SUB) + j
                descs.append(pltpu.async_copy(
                    nf_hbm.at[idx_v.at[g]],
                    rows_v.at[pl.ds(buf * REG + j * SUB, SUB)],
                    gsem,
                ))
            for dsc in descs:
                dsc.wait()
            ocopies.append(pltpu.async_copy(
                rows_v.at[pl.ds(buf * REG, REG)],
                out_hbm.at[pl.ds(wid * ISTEP + r * REG, REG)],
                osems[buf],
            ))
            k += 1
    ocopies[-2].wait()
    ocopies[-1].wait()


@functools.lru_cache(maxsize=None)
def _gather_call(chunk):
    return pl.kernel(
        functools.partial(_gather_body, chunk),
        out_type=(
            jax.ShapeDtypeStruct((CE, D), jnp.float32),
            jax.ShapeDtypeStruct((CE, D), jnp.float32),
        ),
        mesh=_sc_mesh(),
        scratch_types=[
            pltpu.VMEM((2 * ISTEP // SUB, SUB), jnp.int32),
            pltpu.VMEM((2 * REG, D), jnp.float32),
            pltpu.SemaphoreType.DMA,
            pltpu.SemaphoreType.DMA,
            pltpu.SemaphoreType.DMA,
        ],
        name=f"mpn_gather_{chunk}",
    )


def _scatter_body(half, dst_hbm, zeros_hbm, *rest):
    msgs = rest[:SCH]
    part_out = rest[SCH]
    acc_sh, idx_v, rows_v = rest[SCH + 1:]
    c = lax.axis_index("c")
    s = lax.axis_index("s")
    pltpu.sync_copy(zeros_hbm.at[pl.ds(s * ZROWS, ZROWS)],
                    acc_sh.at[pl.ds(s * ZROWS, ZROWS)])
    plsc.subcore_barrier()

    wid = c * NS + s
    for k in range(SCH):
        base_g = (half * SCH + k) * (CE // SUB) + wid * (ISTEP // SUB)
        pltpu.sync_copy(dst_hbm.at[pl.ds(base_g, ISTEP // SUB)], idx_v)
        for h in range(ISTEP // SCHUNK):
            pltpu.sync_copy(
                msgs[k].at[pl.ds(wid * ISTEP + h * SCHUNK, SCHUNK)], rows_v)
            for j in range(SCHUNK // SUB):
                pltpu.sync_copy(
                    rows_v.at[pl.ds(j * SUB, SUB)],
                    acc_sh.at[idx_v.at[h * (SCHUNK // SUB) + j]],
                    add=True,
                )
    plsc.subcore_barrier()
    pltpu.sync_copy(acc_sh.at[pl.ds(s * ZROWS, ZROWS)],
                    part_out.at[c].at[pl.ds(s * ZROWS, ZROWS)])


@functools.lru_cache(maxsize=None)
def _scatter_call(half):
    return pl.kernel(
        functools.partial(_scatter_body, half),
        out_type=jax.ShapeDtypeStruct((NC, NACC, D), jnp.float32),
        mesh=_sc_mesh(),
        scratch_types=[
            pltpu.VMEM_SHARED((NACC, D), jnp.float32),
            pltpu.VMEM((ISTEP // SUB, SUB), jnp.int32),
            pltpu.VMEM((SCHUNK, D), jnp.float32),
        ],
        name=f"mpn_scatter_{half}",
    )


BE = 2048


def _mlp_body(src_ref, dst_ref, ef_ref, w1s_ref, w1d_ref, w1f_ref, b1e_ref,
              w2e_ref, b2e_ref, w1nd_ref, w1nm_ref, b1n_ref, w2n_ref, b2n_ref,
              out_ref):
    bf = jnp.bfloat16
    mm = functools.partial(jax.lax.dot_general,
                           dimension_numbers=(((1,), (0,)), ((), ())),
                           preferred_element_type=jnp.float32)
    src = src_ref[...].astype(bf)
    dst = dst_ref[...].astype(bf)
    ef = ef_ref[...].astype(bf)
    h = (mm(src, w1s_ref[...].astype(bf)) + mm(dst, w1d_ref[...].astype(bf))
         + mm(ef, w1f_ref[...].astype(bf)) + b1e_ref[...])
    h = jnp.maximum(h, 0.0).astype(bf)
    msg = jnp.maximum(mm(h, w2e_ref[...].astype(bf)) + b2e_ref[...], 0.0)
    msgb = msg.astype(bf)
    g = (mm(dst, w1nd_ref[...].astype(bf)) + mm(msgb, w1nm_ref[...].astype(bf))
         + b1n_ref[...])
    g = jnp.maximum(g, 0.0).astype(bf)
    out_ref[...] = jnp.maximum(mm(g, w2n_ref[...].astype(bf)) + b2n_ref[...], 0.0)


def _mlp_call(src_feat, dst_feat, ef, weights):
    eb = lambda i: (i, 0)
    wb = lambda i: (0, 0)
    wb3 = lambda i: (0, 0, 0)
    return pl.pallas_call(
        _mlp_body,
        grid=(CE // BE,),
        in_specs=[
            pl.BlockSpec((BE, D), eb),
            pl.BlockSpec((BE, D), eb),
            pl.BlockSpec((BE, EDGE_DIM), eb),
            pl.BlockSpec((D, 32), wb),
            pl.BlockSpec((D, 32), wb),
            pl.BlockSpec((EDGE_DIM, 32), wb),
            pl.BlockSpec((1, 32), wb),
            pl.BlockSpec((32, D), wb),
            pl.BlockSpec((1, D), wb),
            pl.BlockSpec((D, 64), wb),
            pl.BlockSpec((D, 64), wb),
            pl.BlockSpec((1, 64), wb),
            pl.BlockSpec((64, D), wb),
            pl.BlockSpec((1, D), wb),
        ],
        out_specs=pl.BlockSpec((BE, D), eb),
        out_shape=jax.ShapeDtypeStruct((CE, D), jnp.float32),
        name="mpn_mlp",
    )(src_feat, dst_feat, ef, *weights)


BN = 1000


def _combine_body(nf_ref, pa_ref, pb_ref, out_ref):
    out_ref[...] = (nf_ref[...] + (pa_ref[0, 0] + pa_ref[1, 0])
                    + (pb_ref[0, 0] + pb_ref[1, 0]))


def _combine_call(nf, parts0, parts1):
    return pl.pallas_call(
        _combine_body,
        grid=(N // BN,),
        in_specs=[
            pl.BlockSpec((BN, D), lambda i: (i, 0)),
            pl.BlockSpec((2, 1, BN, D), lambda i: (0, 0, i, 0)),
            pl.BlockSpec((2, 1, BN, D), lambda i: (0, 0, i, 0)),
        ],
        out_specs=pl.BlockSpec((BN, D), lambda i: (i, 0)),
        out_shape=jax.ShapeDtypeStruct((N, D), jnp.float32),
        name="mpn_combine",
    )(nf, parts0.reshape(NC, 1, NACC, D), parts1.reshape(NC, 1, NACC, D))


def kernel(node_features, edge_features, edge_index, W1e, b1e, W2e, b2e,
           W1n, b1n, W2n, b2n):
    src = edge_index[0].astype(jnp.int32)
    dst = edge_index[1].astype(jnp.int32)
    pad = EPAD - E
    ar = jnp.arange(pad, dtype=jnp.int32)
    pad_gather = ar % N
    pad_scatter = N + (ar % NPAD)

    src2d = jnp.concatenate([src, pad_gather]).reshape(EPAD // SUB, SUB)
    dstg2d = jnp.concatenate([dst, pad_gather]).reshape(EPAD // SUB, SUB)
    dsts2d = jnp.concatenate([dst, pad_scatter]).reshape(EPAD // SUB, SUB)
    ef_pad = jnp.concatenate(
        [edge_features, jnp.zeros((pad, EDGE_DIM), jnp.float32)])
    zeros = jnp.zeros((NACC, D), jnp.float32)

    weights = (
        W1e[:D], W1e[D:2 * D], W1e[2 * D:], b1e.reshape(1, 32),
        W2e, b2e.reshape(1, D),
        W1n[:D], W1n[D:], b1n.reshape(1, 64),
        W2n, b2n.reshape(1, D),
    )

    msgs = []
    for c in range(KC):
        src_feat, dst_feat = _gather_call(c)(node_features, src2d, dstg2d)
        ef_c = jax.lax.slice_in_dim(ef_pad, c * CE, (c + 1) * CE)
        msgs.append(_mlp_call(src_feat, dst_feat, ef_c, weights))

    parts0 = _scatter_call(0)(dsts2d, zeros, *msgs[:SCH])
    parts1 = _scatter_call(1)(dsts2d, zeros, *msgs[SCH:])
    return _combine_call(node_features, parts0, parts1)


# speedup vs baseline: 1.1339x; 1.0269x over previous
import functools

import jax
import jax.numpy as jnp
from jax import lax
from jax.experimental import pallas as pl
from jax.experimental.pallas import tpu as pltpu
from jax.experimental.pallas import tpu_sc as plsc

N = 10000
E = 320000
D = 128
EDGE_DIM = 16

NC = 2
NS = 16
NW = NC * NS
SUB = 128
ISTEP = 1024
REG = 256
KC = 10
CE = NW * ISTEP
EPAD = KC * CE
SCH = KC // 2
SCHUNK = 256
NPAD = 240
NACC = N + NPAD
ZROWS = NACC // NS


@functools.lru_cache(maxsize=None)
def _sc_mesh():
    return plsc.VectorSubcoreMesh(core_axis_name="c", subcore_axis_name="s",
                                  num_cores=NC, num_subcores=NS)


def _gather_body(chunk, nf_hbm, src_hbm, dst_hbm, src_out, dst_out,
                 idx_v, rows_v, gsem, osem0, osem1):
    c = lax.axis_index("c")
    s = lax.axis_index("s")
    wid = s * NC + c
    base_g = chunk * (CE // SUB) + wid * (ISTEP // SUB)

    pltpu.sync_copy(src_hbm.at[pl.ds(base_g, ISTEP // SUB)],
                    idx_v.at[pl.ds(0, ISTEP // SUB)])
    pltpu.sync_copy(dst_hbm.at[pl.ds(base_g, ISTEP // SUB)],
                    idx_v.at[pl.ds(ISTEP // SUB, ISTEP // SUB)])

    osems = (osem0, osem1)
    ocopies = []
    k = 0
    for t, out_hbm in ((0, src_out), (1, dst_out)):
        for r in range(ISTEP // REG):
            buf = k % 2
            if k >= 2:
                ocopies[k - 2].wait()
            descs = []
            for j in range(REG // SUB):
                g = t * (ISTEP // SUB) + r * (REG // SUB) + j
                descs.append(pltpu.async_copy(
                    nf_hbm.at[idx_v.at[g]],
                    rows_v.at[pl.ds(buf * REG + j * SUB, SUB)],
                    gsem,
                ))
            for dsc in descs:
                dsc.wait()
            ocopies.append(pltpu.async_copy(
                rows_v.at[pl.ds(buf * REG, REG)],
                out_hbm.at[pl.ds(wid * ISTEP + r * REG, REG)],
                osems[buf],
            ))
            k += 1
    ocopies[-2].wait()
    ocopies[-1].wait()


@functools.lru_cache(maxsize=None)
def _gather_call(chunk):
    return pl.kernel(
        functools.partial(_gather_body, chunk),
        out_type=(
            jax.ShapeDtypeStruct((CE, D), jnp.float32),
            jax.ShapeDtypeStruct((CE, D), jnp.float32),
        ),
        mesh=_sc_mesh(),
        scratch_types=[
            pltpu.VMEM((2 * ISTEP // SUB, SUB), jnp.int32),
            pltpu.VMEM((2 * REG, D), jnp.float32),
            pltpu.SemaphoreType.DMA,
            pltpu.SemaphoreType.DMA,
            pltpu.SemaphoreType.DMA,
        ],
        name=f"mpn_gather_{chunk}",
    )


def _scatter_body(half, dst_hbm, zeros_hbm, *rest):
    msgs = rest[:SCH]
    part_out = rest[SCH]
    acc_sh, idx_v, rows_v = rest[SCH + 1:]
    c = lax.axis_index("c")
    s = lax.axis_index("s")
    pltpu.sync_copy(zeros_hbm.at[pl.ds(s * ZROWS, ZROWS)],
                    acc_sh.at[pl.ds(s * ZROWS, ZROWS)])
    plsc.subcore_barrier()

    wid = c * NS + s
    for k in range(SCH):
        base_g = (half * SCH + k) * (CE // SUB) + wid * (ISTEP // SUB)
        pltpu.sync_copy(dst_hbm.at[pl.ds(base_g, ISTEP // SUB)], idx_v)
        for h in range(ISTEP // SCHUNK):
            pltpu.sync_copy(
                msgs[k].at[pl.ds(wid * ISTEP + h * SCHUNK, SCHUNK)], rows_v)
            for j in range(SCHUNK // SUB):
                pltpu.sync_copy(
                    rows_v.at[pl.ds(j * SUB, SUB)],
                    acc_sh.at[idx_v.at[h * (SCHUNK // SUB) + j]],
                    add=True,
                )
    plsc.subcore_barrier()
    pltpu.sync_copy(acc_sh.at[pl.ds(s * ZROWS, ZROWS)],
                    part_out.at[c].at[pl.ds(s * ZROWS, ZROWS)])


@functools.lru_cache(maxsize=None)
def _scatter_call(half):
    return pl.kernel(
        functools.partial(_scatter_body, half),
        out_type=jax.ShapeDtypeStruct((NC, NACC, D), jnp.float32),
        mesh=_sc_mesh(),
        scratch_types=[
            pltpu.VMEM_SHARED((NACC, D), jnp.float32),
            pltpu.VMEM((ISTEP // SUB, SUB), jnp.int32),
            pltpu.VMEM((SCHUNK, D), jnp.float32),
        ],
        name=f"mpn_scatter_{half}",
    )


BE = 4096


def _mlp_body(src_ref, dst_ref, ef_ref, w1s_ref, w1d_ref, w1f_ref, b1e_ref,
              w2e_ref, b2e_ref, w1nd_ref, w1nm_ref, b1n_ref, w2n_ref, b2n_ref,
              out_ref):
    bf = jnp.bfloat16
    mm = functools.partial(jax.lax.dot_general,
                           dimension_numbers=(((1,), (0,)), ((), ())),
                           preferred_element_type=jnp.float32)
    src = src_ref[...].astype(bf)
    dst = dst_ref[...].astype(bf)
    ef = ef_ref[...].astype(bf)
    h = (mm(src, w1s_ref[...].astype(bf)) + mm(dst, w1d_ref[...].astype(bf))
         + mm(ef, w1f_ref[...].astype(bf)) + b1e_ref[...])
    h = jnp.maximum(h, 0.0).astype(bf)
    msg = jnp.maximum(mm(h, w2e_ref[...].astype(bf)) + b2e_ref[...], 0.0)
    msgb = msg.astype(bf)
    g = (mm(dst, w1nd_ref[...].astype(bf)) + mm(msgb, w1nm_ref[...].astype(bf))
         + b1n_ref[...])
    g = jnp.maximum(g, 0.0).astype(bf)
    out_ref[...] = jnp.maximum(mm(g, w2n_ref[...].astype(bf)) + b2n_ref[...], 0.0)


def _mlp_call(src_feat, dst_feat, ef, weights):
    eb = lambda i: (i, 0)
    wb = lambda i: (0, 0)
    wb3 = lambda i: (0, 0, 0)
    return pl.pallas_call(
        _mlp_body,
        grid=(CE // BE,),
        in_specs=[
            pl.BlockSpec((BE, D), eb),
            pl.BlockSpec((BE, D), eb),
            pl.BlockSpec((BE, EDGE_DIM), eb),
            pl.BlockSpec((D, 32), wb),
            pl.BlockSpec((D, 32), wb),
            pl.BlockSpec((EDGE_DIM, 32), wb),
            pl.BlockSpec((1, 32), wb),
            pl.BlockSpec((32, D), wb),
            pl.BlockSpec((1, D), wb),
            pl.BlockSpec((D, 64), wb),
            pl.BlockSpec((D, 64), wb),
            pl.BlockSpec((1, 64), wb),
            pl.BlockSpec((64, D), wb),
            pl.BlockSpec((1, D), wb),
        ],
        out_specs=pl.BlockSpec((BE, D), eb),
        out_shape=jax.ShapeDtypeStruct((CE, D), jnp.float32),
        name="mpn_mlp",
    )(src_feat, dst_feat, ef, *weights)


BN = 1000


def _combine_body(nf_ref, pa_ref, pb_ref, out_ref):
    out_ref[...] = (nf_ref[...] + (pa_ref[0, 0] + pa_ref[1, 0])
                    + (pb_ref[0, 0] + pb_ref[1, 0]))


def _combine_call(nf, parts0, parts1):
    return pl.pallas_call(
        _combine_body,
        grid=(N // BN,),
        in_specs=[
            pl.BlockSpec((BN, D), lambda i: (i, 0)),
            pl.BlockSpec((2, 1, BN, D), lambda i: (0, 0, i, 0)),
            pl.BlockSpec((2, 1, BN, D), lambda i: (0, 0, i, 0)),
        ],
        out_specs=pl.BlockSpec((BN, D), lambda i: (i, 0)),
        out_shape=jax.ShapeDtypeStruct((N, D), jnp.float32),
        name="mpn_combine",
    )(nf, parts0.reshape(NC, 1, NACC, D), parts1.reshape(NC, 1, NACC, D))


def kernel(node_features, edge_features, edge_index, W1e, b1e, W2e, b2e,
           W1n, b1n, W2n, b2n):
    src = edge_index[0].astype(jnp.int32)
    dst = edge_index[1].astype(jnp.int32)
    pad = EPAD - E
    ar = jnp.arange(pad, dtype=jnp.int32)
    pad_gather = ar % N
    pad_scatter = N + (ar % NPAD)

    src2d = jnp.concatenate([src, pad_gather]).reshape(EPAD // SUB, SUB)
    dstg2d = jnp.concatenate([dst, pad_gather]).reshape(EPAD // SUB, SUB)
    dsts2d = jnp.concatenate([dst, pad_scatter]).reshape(EPAD // SUB, SUB)
    ef_pad = jnp.concatenate(
        [edge_features, jnp.zeros((pad, EDGE_DIM), jnp.float32)])
    zeros = jnp.zeros((NACC, D), jnp.float32)

    weights = (
        W1e[:D], W1e[D:2 * D], W1e[2 * D:], b1e.reshape(1, 32),
        W2e, b2e.reshape(1, D),
        W1n[:D], W1n[D:], b1n.reshape(1, 64),
        W2n, b2n.reshape(1, D),
    )

    msgs = []
    for c in range(KC):
        src_feat, dst_feat = _gather_call(c)(node_features, src2d, dstg2d)
        ef_c = jax.lax.slice_in_dim(ef_pad, c * CE, (c + 1) * CE)
        msgs.append(_mlp_call(src_feat, dst_feat, ef_c, weights))

    parts0 = _scatter_call(0)(dsts2d, zeros, *msgs[:SCH])
    parts1 = _scatter_call(1)(dsts2d, zeros, *msgs[SCH:])
    return _combine_call(node_features, parts0, parts1)
